# baseline (speedup 1.0000x reference)
import functools

import jax
import jax.numpy as jnp
from jax import lax
from jax.experimental import pallas as pl
from jax.experimental.pallas import tpu as pltpu
from jax.experimental.pallas import tpu_sc as plsc

HASH_SZ = 100000
MULT = 31337

NC, NS, L = 2, 16, 16
NW = NC * NS

B, T, D = 8, 2048, 1000
DPB = 1024
DP = DPB // 2
ROWS = B * T
RPW = ROWS // NW
WPB = T // RPW
CH = 32
NCH = RPW // CH


def _body(tokens_hbm, table_hbm, out_hbm,
          tok_v, idx_v, buf0, buf1, gs0, gs1, ss0, ss1):
    cid = lax.axis_index("c")
    sid = lax.axis_index("s")
    wid = sid * NC + cid
    b = wid // WPB
    t0 = (wid % WPB) * RPW
    base = wid * RPW

    pltpu.sync_copy(tokens_hbm.at[pl.ds(b * T, T)], tok_v)

    iota = lax.iota(jnp.int32, L)
    for i in range(RPW // L):
        off = t0 + i * L
        curr = tok_v[pl.ds(off, L)]
        prev = plsc.load_gather(tok_v, [jnp.maximum(iota + (off - 1), 0)])
        idx_v[pl.ds(i * L, L)] = (prev * MULT + curr) % HASH_SZ

    def g_start(j, buf, sem):
        return pltpu.async_copy(
            table_hbm.at[idx_v.at[pl.ds(j * CH, CH)]], buf, sem)

    def s_start(j, buf, sem):
        return pltpu.async_copy(
            buf, out_hbm.at[pl.ds(base + j * CH, CH)], sem)

    bufs = (buf0, buf1)
    gsems = (gs0, gs1)
    ssems = (ss0, ss1)
    zero = jnp.zeros((L,), jnp.float32)
    g = [None, None]
    s = [None, None]

    g[0] = g_start(0, bufs[0], gsems[0])
    for j in range(NCH):
        p = j & 1
        g[p].wait()
        if j == 0:
            @pl.when(t0 == 0)
            def _zero_row():
                for k in range(DP // L):
                    bufs[0][0, pl.ds(k * L, L)] = zero
        s[p] = s_start(j, bufs[p], ssems[p])
        if j + 1 < NCH:
            if j >= 1:
                s[1 - p].wait()
            g[1 - p] = g_start(j + 1, bufs[1 - p], gsems[1 - p])
    s[0].wait()
    s[1].wait()


@functools.cache
def _gather_call():
    return pl.kernel(
        _body,
        out_type=jax.ShapeDtypeStruct((ROWS, DP), jnp.float32),
        mesh=plsc.VectorSubcoreMesh(
            core_axis_name="c", subcore_axis_name="s",
            num_cores=NC, num_subcores=NS),
        scratch_types=[
            pltpu.VMEM((T,), jnp.int32),
            pltpu.VMEM((RPW,), jnp.int32),
            pltpu.VMEM((CH, DP), jnp.float32),
            pltpu.VMEM((CH, DP), jnp.float32),
            pltpu.SemaphoreType.DMA,
            pltpu.SemaphoreType.DMA,
            pltpu.SemaphoreType.DMA,
            pltpu.SemaphoreType.DMA,
        ],
        compiler_params=pltpu.CompilerParams(
            needs_layout_passes=False, use_tc_tiling_on_sc=True),
    )


def kernel(tokens, table):
    tb = jnp.pad(table.astype(jnp.bfloat16), ((0, 0), (0, DPB - D)))
    tp = lax.bitcast_convert_type(
        tb.reshape(HASH_SZ, DP, 2), jnp.float32)
    out = _gather_call()(tokens.reshape(-1), tp)
    ob = lax.bitcast_convert_type(out, jnp.bfloat16).reshape(ROWS, DPB)
    return ob[:, :D].astype(jnp.float32).reshape(B, T, D)


# speedup vs baseline: 3.2254x; 3.2254x over previous
import functools

import jax
import jax.numpy as jnp
from jax import lax
from jax.experimental import pallas as pl
from jax.experimental.pallas import tpu as pltpu
from jax.experimental.pallas import tpu_sc as plsc

HASH_SZ = 100000
MULT = 31337

NC, NS, L = 2, 16, 16
NW = NC * NS

B, T, D = 8, 2048, 1000
DPB = 1024
DP = DPB // 2
ROWS = B * T
RPW = ROWS // NW
WPB = T // RPW
CH = 32
NCH = RPW // CH


RB = 1000


def _pack_body(x_ref, o_ref):
    u_lo = lax.bitcast_convert_type(x_ref[:, :DP], jnp.int32)
    x_hi = jnp.concatenate(
        [x_ref[:, DP:], jnp.zeros((RB, DPB - D), jnp.float32)], axis=1)
    u_hi = lax.bitcast_convert_type(x_hi, jnp.int32)
    word = lax.shift_right_logical(u_lo, 16) | (u_hi & jnp.int32(-65536))
    o_ref[...] = lax.bitcast_convert_type(word, jnp.float32)


@functools.cache
def _pack_call():
    return pl.pallas_call(
        _pack_body,
        grid=(HASH_SZ // RB,),
        in_specs=[pl.BlockSpec((RB, D), lambda i: (i, 0))],
        out_specs=pl.BlockSpec((RB, DP), lambda i: (i, 0)),
        out_shape=jax.ShapeDtypeStruct((HASH_SZ, DP), jnp.float32),
    )


def _body(tokens_hbm, table_hbm, out_hbm,
          tok_v, idx_v, buf0, buf1, gs0, gs1, ss0, ss1):
    cid = lax.axis_index("c")
    sid = lax.axis_index("s")
    wid = sid * NC + cid
    b = wid // WPB
    t0 = (wid % WPB) * RPW
    base = wid * RPW

    pltpu.sync_copy(tokens_hbm.at[pl.ds(b * T, T)], tok_v)

    iota = lax.iota(jnp.int32, L)
    for i in range(RPW // L):
        off = t0 + i * L
        curr = tok_v[pl.ds(off, L)]
        prev = plsc.load_gather(tok_v, [jnp.maximum(iota + (off - 1), 0)])
        idx_v[pl.ds(i * L, L)] = (prev * MULT + curr) % HASH_SZ

    def g_start(j, buf, sem):
        return pltpu.async_copy(
            table_hbm.at[idx_v.at[pl.ds(j * CH, CH)]], buf, sem)

    def s_start(j, buf, sem):
        return pltpu.async_copy(
            buf, out_hbm.at[pl.ds(base + j * CH, CH)], sem)

    bufs = (buf0, buf1)
    gsems = (gs0, gs1)
    ssems = (ss0, ss1)
    zero = jnp.zeros((L,), jnp.float32)
    g = [None, None]
    s = [None, None]

    g[0] = g_start(0, bufs[0], gsems[0])
    for j in range(NCH):
        p = j & 1
        g[p].wait()
        if j == 0:
            @pl.when(t0 == 0)
            def _zero_row():
                for k in range(DP // L):
                    bufs[0][0, pl.ds(k * L, L)] = zero
        s[p] = s_start(j, bufs[p], ssems[p])
        if j + 1 < NCH:
            if j >= 1:
                s[1 - p].wait()
            g[1 - p] = g_start(j + 1, bufs[1 - p], gsems[1 - p])
    s[0].wait()
    s[1].wait()


@functools.cache
def _gather_call():
    return pl.kernel(
        _body,
        out_type=jax.ShapeDtypeStruct((ROWS, DP), jnp.float32),
        mesh=plsc.VectorSubcoreMesh(
            core_axis_name="c", subcore_axis_name="s",
            num_cores=NC, num_subcores=NS),
        scratch_types=[
            pltpu.VMEM((T,), jnp.int32),
            pltpu.VMEM((RPW,), jnp.int32),
            pltpu.VMEM((CH, DP), jnp.float32),
            pltpu.VMEM((CH, DP), jnp.float32),
            pltpu.SemaphoreType.DMA,
            pltpu.SemaphoreType.DMA,
            pltpu.SemaphoreType.DMA,
            pltpu.SemaphoreType.DMA,
        ],
        compiler_params=pltpu.CompilerParams(
            needs_layout_passes=False, use_tc_tiling_on_sc=True),
    )


def kernel(tokens, table):
    tp = _pack_call()(table)
    out = _gather_call()(tokens.reshape(-1), tp)
    u = lax.bitcast_convert_type(out, jnp.int32)
    f_lo = lax.bitcast_convert_type(u << 16, jnp.float32)
    f_hi = lax.bitcast_convert_type(u & jnp.int32(-65536), jnp.float32)
    full = jnp.concatenate([f_lo, f_hi], axis=1)
    return full[:, :D].reshape(B, T, D)


# speedup vs baseline: 4.8906x; 1.5163x over previous
import functools

import jax
import jax.numpy as jnp
from jax import lax
from jax.experimental import pallas as pl
from jax.experimental.pallas import tpu as pltpu
from jax.experimental.pallas import tpu_sc as plsc

HASH_SZ = 100000
MULT = 31337

NC, NS, L = 2, 16, 16
NW = NC * NS

B, T, D = 8, 2048, 1000
ROWS = B * T
RPW = ROWS // NW
WPB = T // RPW
CH = 32
NCH = RPW // CH


def _body(tokens_hbm, table_hbm, out_hbm,
          tok_v, idx_v, buf_v, gsem, ssem):
    cid = lax.axis_index("c")
    sid = lax.axis_index("s")
    wid = sid * NC + cid
    b = wid // WPB
    t0 = (wid % WPB) * RPW
    base = wid * RPW

    pltpu.sync_copy(tokens_hbm.at[pl.ds(b * T, T)], tok_v)

    iota = lax.iota(jnp.int32, L)
    for i in range(RPW // L):
        off = t0 + i * L
        curr = tok_v[pl.ds(off, L)]
        prev = plsc.load_gather(tok_v, [jnp.maximum(iota + (off - 1), 0)])
        idx_v[pl.ds(i * L, L)] = (prev * MULT + curr) % HASH_SZ

    zero = jnp.zeros((L,), jnp.float32)

    def chunk(j, _):
        p = lax.rem(j, 2)

        @pl.when(j >= 2)
        def _drain():
            pltpu.make_async_copy(
                buf_v.at[p],
                out_hbm.at[pl.ds(base + (j - 2) * CH, CH)],
                ssem,
            ).wait()

        for rr in range(CH):
            if rr % L == 0:
                grp = idx_v[pl.ds(j * CH + rr, L)]
            row = grp[rr % L]
            pltpu.async_copy(
                table_hbm.at[pl.ds(row, 1)],
                buf_v.at[p, pl.ds(rr, 1)],
                gsem,
            )
        pltpu.make_async_copy(
            table_hbm.at[pl.ds(0, CH)],
            buf_v.at[p],
            gsem,
        ).wait()

        @pl.when((j == 0) & (t0 == 0))
        def _zero_row():
            for k in range(D // L):
                buf_v[p, 0, pl.ds(k * L, L)] = zero
            buf_v[p, 0, pl.ds(D - L, L)] = zero

        pltpu.async_copy(
            buf_v.at[p],
            out_hbm.at[pl.ds(base + j * CH, CH)],
            ssem,
        )
        return 0

    lax.fori_loop(0, NCH, chunk, 0, unroll=False)

    for j in (NCH - 2, NCH - 1):
        pltpu.make_async_copy(
            buf_v.at[j % 2],
            out_hbm.at[pl.ds(base + j * CH, CH)],
            ssem,
        ).wait()


@functools.cache
def _gather_call():
    return pl.kernel(
        _body,
        out_type=jax.ShapeDtypeStruct((ROWS, D), jnp.float32),
        mesh=plsc.VectorSubcoreMesh(
            core_axis_name="c", subcore_axis_name="s",
            num_cores=NC, num_subcores=NS),
        scratch_types=[
            pltpu.VMEM((T,), jnp.int32),
            pltpu.VMEM((RPW,), jnp.int32),
            pltpu.VMEM((2, CH, D), jnp.float32),
            pltpu.SemaphoreType.DMA,
            pltpu.SemaphoreType.DMA,
        ],
        compiler_params=pltpu.CompilerParams(
            needs_layout_passes=False, use_tc_tiling_on_sc=True),
    )


def kernel(tokens, table):
    out = _gather_call()(tokens.reshape(-1), table)
    return out.reshape(B, T, D)
